# baseline (speedup 1.0000x reference)
import functools

import jax
import jax.numpy as jnp
import numpy as np
from jax import lax
from jax.experimental import pallas as pl

_N = 10000
_E = 160000
_E2 = 163840
_BE = 1024
_BN = 1000
_NEB = _E2 // _BE
_NNB = _N // _BN
_F32 = jnp.float32
_HIGH = lax.Precision.HIGHEST


def _dot(a, b):
    return lax.dot_general(a, b, (((1,), (0,)), ((), ())),
                           precision=_HIGH, preferred_element_type=_F32)


def _dott(a, b):
    return lax.dot_general(a, b, (((0,), (0,)), ((), ())),
                           precision=_HIGH, preferred_element_type=_F32)


def _stats_body(h_ref, o_ref):
    i = pl.program_id(0)

    @pl.when(i == 0)
    def _():
        o_ref[...] = jnp.zeros_like(o_ref)

    blk = h_ref[...]
    o_ref[0:1, :] += jnp.sum(blk, axis=0, keepdims=True)
    o_ref[1:2, :] += jnp.sum(blk * blk, axis=0, keepdims=True)


def _stats(h):
    n, d = h.shape
    return pl.pallas_call(
        _stats_body,
        grid=(n // 400,),
        in_specs=[pl.BlockSpec((400, d), lambda i: (i, 0))],
        out_specs=pl.BlockSpec((8, d), lambda i: (0, 0)),
        out_shape=jax.ShapeDtypeStruct((8, d), _F32),
    )(h)


def _bn_body(h_ref, st_ref, g_ref, b_ref, o_ref):
    m = st_ref[0:1, :] / _N
    var = st_ref[1:2, :] / _N - m * m
    alpha = g_ref[...] * lax.rsqrt(var + 1e-5)
    o_ref[...] = h_ref[...] * alpha + (b_ref[...] - m * alpha)


def _bn_apply(h, st, g, b):
    n, d = h.shape
    return pl.pallas_call(
        _bn_body,
        grid=(n // 400,),
        in_specs=[
            pl.BlockSpec((400, d), lambda i: (i, 0)),
            pl.BlockSpec((8, d), lambda i: (0, 0)),
            pl.BlockSpec((1, d), lambda i: (0, 0)),
            pl.BlockSpec((1, d), lambda i: (0, 0)),
        ],
        out_specs=pl.BlockSpec((400, d), lambda i: (i, 0)),
        out_shape=jax.ShapeDtypeStruct((n, d), _F32),
    )(h, st, g.reshape(1, d), b.reshape(1, d))


def _mm_body(a_ref, w_ref, o_ref):
    o_ref[...] = _dot(a_ref[...], w_ref[...])


def _mm(a, w, blk_n, blk_m):
    n, k = a.shape
    m = w.shape[1]
    return pl.pallas_call(
        _mm_body,
        grid=(n // blk_n, m // blk_m),
        in_specs=[
            pl.BlockSpec((blk_n, k), lambda i, j: (i, 0)),
            pl.BlockSpec((k, blk_m), lambda i, j: (0, j)),
        ],
        out_specs=pl.BlockSpec((blk_n, blk_m), lambda i, j: (i, j)),
        out_shape=jax.ShapeDtypeStruct((n, m), _F32),
    )(a, w)


def _oh(idx_ref, base):
    cols = lax.broadcasted_iota(jnp.int32, (_BE, _BN), 1) + base
    return jnp.where(idx_ref[...] == cols, 1.0, 0.0)


def _gath2_body(si_ref, di_ref, ts_ref, td_ref, gs_ref, gd_ref):
    j = pl.program_id(1)

    @pl.when(j == 0)
    def _():
        gs_ref[...] = jnp.zeros_like(gs_ref)
        gd_ref[...] = jnp.zeros_like(gd_ref)

    base = j * _BN
    gs_ref[...] += _dot(_oh(si_ref, base), ts_ref[...])
    gd_ref[...] += _dot(_oh(di_ref, base), td_ref[...])


def _gath2(src2, dst2, ts, td):
    return pl.pallas_call(
        _gath2_body,
        grid=(_NEB, _NNB),
        in_specs=[
            pl.BlockSpec((_BE, 1), lambda i, j: (i, 0)),
            pl.BlockSpec((_BE, 1), lambda i, j: (i, 0)),
            pl.BlockSpec((_BN, 8), lambda i, j: (j, 0)),
            pl.BlockSpec((_BN, 8), lambda i, j: (j, 0)),
        ],
        out_specs=[
            pl.BlockSpec((_BE, 8), lambda i, j: (i, 0)),
            pl.BlockSpec((_BE, 8), lambda i, j: (i, 0)),
        ],
        out_shape=[jax.ShapeDtypeStruct((_E2, 8), _F32)] * 2,
    )(src2, dst2, ts, td)


def _w12_body(c0, invs, gs_ref, gd_ref, ae_ref, w_ref):
    ae = ae_ref[...]
    z = gs_ref[...][:, 0:4] + gd_ref[...][:, 0:4] + ae[:, c0:c0 + 4]
    lg = jnp.maximum(z, 0.2 * z) * invs + ae[:, c0 + 4:c0 + 8]
    w4 = jnp.exp(lg)
    w_ref[...] = jnp.concatenate([w4, jnp.zeros_like(w4)], axis=1)


def _w12(gs, gd, ae, c0):
    return pl.pallas_call(
        functools.partial(_w12_body, c0, 1.0 / 16.0),
        grid=(_NEB,),
        in_specs=[
            pl.BlockSpec((_BE, 8), lambda i: (i, 0)),
            pl.BlockSpec((_BE, 8), lambda i: (i, 0)),
            pl.BlockSpec((_BE, 48), lambda i: (i, 0)),
        ],
        out_specs=pl.BlockSpec((_BE, 8), lambda i: (i, 0)),
        out_shape=jax.ShapeDtypeStruct((_E2, 8), _F32),
    )(gs, gd, ae)


def _w3_body(invs, gs_ref, gd_ref, ae_ref, w_ref):
    ae = ae_ref[...]
    gs = gs_ref[...]
    z = gs[:, 0:1] + gd_ref[...][:, 0:1] + ae[:, 32:33]
    lg = jnp.maximum(z, 0.2 * z) * invs + ae[:, 33:34]
    w = jnp.exp(lg)
    w_ref[...] = jnp.concatenate(
        [w, w * gs[:, 1:2], w * gs[:, 2:3], jnp.zeros((_BE, 5), _F32)], axis=1)


def _w3(gs, gd, ae):
    return pl.pallas_call(
        functools.partial(_w3_body, float(1.0 / np.sqrt(2.0))),
        grid=(_NEB,),
        in_specs=[
            pl.BlockSpec((_BE, 8), lambda i: (i, 0)),
            pl.BlockSpec((_BE, 8), lambda i: (i, 0)),
            pl.BlockSpec((_BE, 48), lambda i: (i, 0)),
        ],
        out_specs=pl.BlockSpec((_BE, 8), lambda i: (i, 0)),
        out_shape=jax.ShapeDtypeStruct((_E2, 8), _F32),
    )(gs, gd, ae)


def _scat80_body(di_ref, w_ref, ef_ref, o_ref):
    k = pl.program_id(1)

    @pl.when(k == 0)
    def _():
        o_ref[...] = jnp.zeros_like(o_ref)

    w = w_ref[...]
    ef = ef_ref[...]
    pay = jnp.concatenate(
        [w[:, 0:4], jnp.zeros((_BE, 12), _F32),
         w[:, 0:1] * ef, w[:, 1:2] * ef, w[:, 2:3] * ef, w[:, 3:4] * ef],
        axis=1)
    o_ref[...] += _dott(_oh(di_ref, pl.program_id(0) * _BN), pay)


def _scat80(dst2, w, ef):
    return pl.pallas_call(
        _scat80_body,
        grid=(_NNB, _NEB),
        in_specs=[
            pl.BlockSpec((_BE, 1), lambda i, k: (k, 0)),
            pl.BlockSpec((_BE, 8), lambda i, k: (k, 0)),
            pl.BlockSpec((_BE, 16), lambda i, k: (k, 0)),
        ],
        out_specs=pl.BlockSpec((_BN, 80), lambda i, k: (i, 0)),
        out_shape=jax.ShapeDtypeStruct((_N, 80), _F32),
    )(dst2, w, ef)


def _scat32_body(di_ref, w_ref, ef_ref, o_ref):
    k = pl.program_id(1)

    @pl.when(k == 0)
    def _():
        o_ref[...] = jnp.zeros_like(o_ref)

    w = w_ref[...]
    pay = jnp.concatenate(
        [w[:, 0:1] * ef_ref[...], w[:, 1:3], w[:, 0:1],
         jnp.zeros((_BE, 13), _F32)], axis=1)
    o_ref[...] += _dott(_oh(di_ref, pl.program_id(0) * _BN), pay)


def _scat32(dst2, w, ef):
    return pl.pallas_call(
        _scat32_body,
        grid=(_NNB, _NEB),
        in_specs=[
            pl.BlockSpec((_BE, 1), lambda i, k: (k, 0)),
            pl.BlockSpec((_BE, 8), lambda i, k: (k, 0)),
            pl.BlockSpec((_BE, 16), lambda i, k: (k, 0)),
        ],
        out_specs=pl.BlockSpec((_BN, 32), lambda i, k: (i, 0)),
        out_shape=jax.ShapeDtypeStruct((_N, 32), _F32),
    )(dst2, w, ef)


def _gathp_body(si_ref, w_ref, hp_ref, g_ref):
    j = pl.program_id(1)

    @pl.when(j == 0)
    def _():
        g_ref[...] = jnp.zeros_like(g_ref)

    g_ref[...] += _dot(_oh(si_ref, j * _BN), hp_ref[...])

    @pl.when(j == _NNB - 1)
    def _():
        head = lax.broadcasted_iota(jnp.int32, (_BE, 1024), 1) // 256
        w = w_ref[...]
        scale = (jnp.where(head == 0, w[:, 0:1], 0.0)
                 + jnp.where(head == 1, w[:, 1:2], 0.0)
                 + jnp.where(head == 2, w[:, 2:3], 0.0)
                 + jnp.where(head == 3, w[:, 3:4], 0.0))
        g_ref[...] *= scale


def _gathp(src2, w, hp):
    return pl.pallas_call(
        _gathp_body,
        grid=(_NEB, _NNB),
        in_specs=[
            pl.BlockSpec((_BE, 1), lambda i, j: (i, 0)),
            pl.BlockSpec((_BE, 8), lambda i, j: (i, 0)),
            pl.BlockSpec((_BN, 1024), lambda i, j: (j, 0)),
        ],
        out_specs=pl.BlockSpec((_BE, 1024), lambda i, j: (i, 0)),
        out_shape=jax.ShapeDtypeStruct((_E2, 1024), _F32),
    )(src2, w, hp)


def _scat1024_body(di_ref, g_ref, o_ref):
    k = pl.program_id(1)

    @pl.when(k == 0)
    def _():
        o_ref[...] = jnp.zeros_like(o_ref)

    o_ref[...] += _dott(_oh(di_ref, pl.program_id(0) * _BN), g_ref[...])


def _scat1024(dst2, g):
    return pl.pallas_call(
        _scat1024_body,
        grid=(_NNB, _NEB),
        in_specs=[
            pl.BlockSpec((_BE, 1), lambda i, k: (k, 0)),
            pl.BlockSpec((_BE, 1024), lambda i, k: (k, 0)),
        ],
        out_specs=pl.BlockSpec((_BN, 1024), lambda i, k: (i, 0)),
        out_shape=jax.ShapeDtypeStruct((_N, 1024), _F32),
    )(dst2, g)


def _red_body(a_ref, s_ref, d_ref):
    r = a_ref[...]
    dinv = 1.0 / (r[:, 0:4] + 1e-16)
    d_ref[...] = jnp.concatenate([dinv, jnp.zeros_like(dinv)], axis=1)
    s_ref[...] = r[:, 16:80]


def _reduce80(acc):
    return pl.pallas_call(
        _red_body,
        grid=(_N // 400,),
        in_specs=[pl.BlockSpec((400, 80), lambda i: (i, 0))],
        out_specs=[
            pl.BlockSpec((400, 64), lambda i: (i, 0)),
            pl.BlockSpec((400, 8), lambda i: (i, 0)),
        ],
        out_shape=[
            jax.ShapeDtypeStruct((_N, 64), _F32),
            jax.ShapeDtypeStruct((_N, 8), _F32),
        ],
    )(acc)


def _combine_body(z_ref, m_ref, o2_ref, dinv_ref, b_ref, o_ref):
    c = pl.program_id(1)
    h = c // 2
    lanes = lax.broadcasted_iota(jnp.int32, (1, 8), 1)
    sel = jnp.where(lanes == h, 1.0, 0.0)
    dinv = jnp.sum(dinv_ref[...] * sel, axis=1, keepdims=True)
    rows = lax.broadcasted_iota(jnp.int32, (8, 1), 0)
    rsel = jnp.where(rows == c, 1.0, 0.0)
    bias = jnp.sum(b_ref[...] * rsel, axis=0, keepdims=True)
    g = (m_ref[...] + o2_ref[...]) * dinv + bias
    o_ref[...] = jnp.maximum(z_ref[...] + g, 0.0)


def _combine(big, m, o2, dinv, bias):
    return pl.pallas_call(
        _combine_body,
        grid=(_N // 400, 8),
        in_specs=[
            pl.BlockSpec((400, 128), lambda i, c: (i, c + 1)),
            pl.BlockSpec((400, 128), lambda i, c: (i, c)),
            pl.BlockSpec((400, 128), lambda i, c: (i, c)),
            pl.BlockSpec((400, 8), lambda i, c: (i, 0)),
            pl.BlockSpec((8, 128), lambda i, c: (0, 0)),
        ],
        out_specs=pl.BlockSpec((400, 128), lambda i, c: (i, c)),
        out_shape=jax.ShapeDtypeStruct((_N, 1024), _F32),
    )(big, m, o2, dinv, bias)


def _combine3_body(z_ref, a_ref, we_ref, b_ref, o_ref):
    r = a_ref[...]
    o2 = _dot(r, we_ref[...])
    dinv = 1.0 / (r[:, 18:19] + 1e-16)
    g = o2 * dinv + b_ref[...]
    o_ref[...] = jnp.maximum(z_ref[...] + g, 0.0)


def _combine3(w3out, acc3, we_ext, b3p):
    return pl.pallas_call(
        _combine3_body,
        grid=(_N // 400,),
        in_specs=[
            pl.BlockSpec((400, 128), lambda i: (i, 2)),
            pl.BlockSpec((400, 32), lambda i: (i, 0)),
            pl.BlockSpec((32, 128), lambda i: (0, 0)),
            pl.BlockSpec((1, 128), lambda i: (0, 0)),
        ],
        out_specs=pl.BlockSpec((400, 128), lambda i: (i, 0)),
        out_shape=jax.ShapeDtypeStruct((_N, 128), _F32),
    )(w3out, acc3, we_ext, b3p)


def _att_fold(wg, att):
    k = wg.shape[0]
    h, o = att.shape
    return jnp.einsum("khc,hc->kh", wg.reshape(k, h, o), att)


def kernel(x, edge_index, edge_attr, params):
    p = params
    g1, g2, g3 = p["gat1"], p["gat2"], p["gat3"]

    pad = _E2 - _E
    srcp = jnp.concatenate([edge_index[0], jnp.zeros((pad,), jnp.int32)])
    dstp = jnp.concatenate(
        [edge_index[1], jnp.full((pad,), _N, jnp.int32)])
    src2 = srcp.reshape(_E2, 1)
    dst2 = dstp.reshape(_E2, 1)
    ea17 = jnp.concatenate(
        [edge_attr[:, 0:17], jnp.zeros((pad, 17), _F32)], axis=0)
    efeat = ea17[:, 1:17]

    def big_w(g, lin_w, k):
        asrc = _att_fold(g["W"], g["att_src"])
        adst = _att_fold(g["W"], g["att_dst"])
        return jnp.concatenate([
            asrc, jnp.zeros((k, 12), _F32),
            adst, jnp.zeros((k, 108), _F32),
            lin_w, g["W"],
        ], axis=1)

    w_all1 = big_w(g1, p["lin1_W"], 128)
    w_all2 = big_w(g2, p["lin2_W"], 1024)

    def ve_cols(g):
        return _att_fold(g["We"], g["att_e"])

    m17 = jnp.zeros((17, 48), _F32)
    m17 = m17.at[1:17, 0:4].set(ve_cols(g1))
    m17 = m17.at[0, 4:8].set(g1["w_time"])
    m17 = m17.at[1:17, 16:20].set(ve_cols(g2))
    m17 = m17.at[0, 20:24].set(g2["w_time"])
    m17 = m17.at[1:17, 32:33].set(ve_cols(g3))
    m17 = m17.at[0, 33].set(g3["w_time"][0])

    def wblk(g):
        return jax.scipy.linalg.block_diag(
            *[g["We"][:, h * 256:(h + 1) * 256] for h in range(4)])

    wblk1, wblk2 = wblk(g1), wblk(g2)
    bias1 = (p["lin1_b"] + g1["bias"]).reshape(8, 128)
    bias2 = (p["lin2_b"] + g2["bias"]).reshape(8, 128)

    w3all = jnp.zeros((1024, 384), _F32)
    w3all = w3all.at[:, 0:1].set(_att_fold(g3["W"], g3["att_src"]))
    w3all = w3all.at[:, 1:3].set(g3["W"])
    w3all = w3all.at[:, 128:129].set(_att_fold(g3["W"], g3["att_dst"]))
    w3all = w3all.at[:, 256:258].set(p["lin3_W"])

    we_ext = jnp.zeros((32, 128), _F32)
    we_ext = we_ext.at[0:16, 0:2].set(g3["We"])
    we_ext = we_ext.at[16, 0].set(1.0)
    we_ext = we_ext.at[17, 1].set(1.0)
    b3p = jnp.zeros((1, 128), _F32)
    b3p = b3p.at[0, 0:2].set(p["lin3_b"] + g3["bias"])

    aetw = _mm(ea17, m17, 2048, 48)

    st1 = _stats(x)
    hb1 = _bn_apply(x, st1, p["bn1_g"], p["bn1_b"])
    big1 = _mm(hb1, w_all1, 400, 128)
    gs1, gd1 = _gath2(src2, dst2, big1[:, 0:8], big1[:, 16:24])
    w1 = _w12(gs1, gd1, aetw, 0)
    acc1 = _scat80(dst2, w1, efeat)
    s1, dinv1 = _reduce80(acc1)
    m1 = _scat1024(dst2, _gathp(src2, w1, big1[:, 1152:2176]))
    out2_1 = _mm(s1, wblk1, 400, 128)
    h2 = _combine(big1, m1, out2_1, dinv1, bias1)

    st2 = _stats(h2)
    hb2 = _bn_apply(h2, st2, p["bn2_g"], p["bn2_b"])
    big2 = _mm(hb2, w_all2, 400, 128)
    gs2, gd2 = _gath2(src2, dst2, big2[:, 0:8], big2[:, 16:24])
    w2 = _w12(gs2, gd2, aetw, 16)
    acc2 = _scat80(dst2, w2, efeat)
    s2, dinv2 = _reduce80(acc2)
    m2 = _scat1024(dst2, _gathp(src2, w2, big2[:, 1152:2176]))
    out2_2 = _mm(s2, wblk2, 400, 128)
    h3 = _combine(big2, m2, out2_2, dinv2, bias2)

    st3 = _stats(h3)
    hb3 = _bn_apply(h3, st3, p["bn3_g"], p["bn3_b"])
    w3out = _mm(hb3, w3all, 400, 128)
    gs3, gd3 = _gath2(src2, dst2, w3out[:, 0:8], w3out[:, 128:136])
    w3 = _w3(gs3, gd3, aetw)
    acc3 = _scat32(dst2, w3, efeat)
    out128 = _combine3(w3out, acc3, we_ext, b3p)
    return out128[:, 0:2]
